# Initial kernel scaffold; baseline (speedup 1.0000x reference)
import jax
import jax.numpy as jnp
from jax.experimental import pallas as pl


def kernel(boxes, scores):
    raise NotImplementedError("write your pallas kernel here")


# speedup vs baseline: 406.8468x; 406.8468x over previous
import jax
import jax.numpy as jnp
from jax import lax
from jax.experimental import pallas as pl

_CONF = 0.5
_MIN_SZ = 0.01
_MIN_AR = 0.2
_MAX_AR = 5.0
_NMS_T = 0.5
_MAXDET = 100
_NEG = -1e9
_CUT = -1e8

_ROWS, _COLS = 8, 640
_PAD = _ROWS * _COLS


def _nms_body(coords_ref, sc_ref, out_ref):
    x1 = jnp.clip(coords_ref[0], 0.0, 1.0)
    y1 = jnp.clip(coords_ref[1], 0.0, 1.0)
    x2 = jnp.clip(coords_ref[2], 0.0, 1.0)
    y2 = jnp.clip(coords_ref[3], 0.0, 1.0)
    s = sc_ref[...]

    w = x2 - x1
    h = y2 - y1
    valid = (s > _CONF) & (w > _MIN_SZ) & (h > _MIN_SZ)
    aspect = w / (h + 1e-6)
    valid = valid & (aspect > _MIN_AR) & (aspect < _MAX_AR)

    x1 = jnp.where(valid, x1, 0.0)
    y1 = jnp.where(valid, y1, 0.0)
    x2 = jnp.where(valid, x2, 0.0)
    y2 = jnp.where(valid, y2, 0.0)
    area = (x2 - x1) * (y2 - y1)

    s_act = jnp.where(valid, s, _NEG)
    s_sup = jnp.full_like(s, _NEG)

    row_i = lax.broadcasted_iota(jnp.int32, (_ROWS, _COLS), 0)
    col_i = lax.broadcasted_iota(jnp.int32, (_ROWS, _COLS), 1)
    flat = row_i * _COLS + col_i
    big = jnp.int32(1 << 30)

    def body(t, carry):
        s_act, s_sup = carry
        m1 = jnp.max(s_act)
        use1 = m1 > _CUT
        m2 = jnp.max(s_sup)
        use2 = jnp.logical_and(jnp.logical_not(use1), m2 > _CUT)

        pool = jnp.where(use1, s_act, s_sup)
        target = jnp.where(use1, m1, m2)
        idx = jnp.min(jnp.where(pool == target, flat, big))
        m = flat == idx

        sx1 = jnp.sum(jnp.where(m, x1, 0.0))
        sy1 = jnp.sum(jnp.where(m, y1, 0.0))
        sx2 = jnp.sum(jnp.where(m, x2, 0.0))
        sy2 = jnp.sum(jnp.where(m, y2, 0.0))
        sarea = jnp.sum(jnp.where(m, area, 0.0))

        any_sel = jnp.logical_or(use1, use2)
        out_score = jnp.where(use1, m1, _NEG)
        row = jnp.concatenate(
            [
                jnp.where(any_sel, sx1, 0.0).reshape(1, 1),
                jnp.where(any_sel, sy1, 0.0).reshape(1, 1),
                jnp.where(any_sel, sx2, 0.0).reshape(1, 1),
                jnp.where(any_sel, sy2, 0.0).reshape(1, 1),
                out_score.reshape(1, 1),
            ],
            axis=1,
        )
        out_ref[pl.ds(t, 1), :] = row

        ix1 = jnp.maximum(x1, sx1)
        iy1 = jnp.maximum(y1, sy1)
        ix2 = jnp.minimum(x2, sx2)
        iy2 = jnp.minimum(y2, sy2)
        inter = jnp.maximum(ix2 - ix1, 0.0) * jnp.maximum(iy2 - iy1, 0.0)
        union = area + sarea - inter
        iou = inter / (union + 1e-9)
        ov = iou > _NMS_T

        newly_sup = use1 & ov & jnp.logical_not(m) & (s_act > _CUT)
        s_sup = jnp.where(newly_sup, s, jnp.where(jnp.logical_and(use2, m), _NEG, s_sup))
        s_act = jnp.where(jnp.logical_and(use1, jnp.logical_or(ov, m)), _NEG, s_act)
        return s_act, s_sup

    lax.fori_loop(0, _MAXDET, body, (s_act, s_sup))


def kernel(boxes, scores):
    n = boxes.shape[0]
    boxes_p = jnp.zeros((_PAD, 4), jnp.float32).at[:n].set(boxes)
    scores_p = jnp.full((_PAD,), -1.0, jnp.float32).at[:n].set(scores)
    coords = boxes_p.T.reshape(4, _ROWS, _COLS)
    sc2d = scores_p.reshape(_ROWS, _COLS)
    return pl.pallas_call(
        _nms_body,
        out_shape=jax.ShapeDtypeStruct((_MAXDET, 5), jnp.float32),
    )(coords, sc2d)
